# baseline (speedup 1.0000x reference)
import jax
import jax.numpy as jnp
from jax.experimental import pallas as pl
from jax.experimental.pallas import tpu as pltpu

_K, _N, _D = 8, 2048, 128
_BI = 128
_BJ = 1024


def _fused_kernel(theta_ref, T_ref, a_ref, x_ref, alpha_ref, W_ref, b_ref,
                  o_ref, acc_ref):
    j = pl.program_id(1)
    nj = pl.num_programs(1)

    q = theta_ref[0] * T_ref[0]
    for k in range(1, _K):
        q += theta_ref[k] * T_ref[k]
    q *= a_ref[...]

    partial = jnp.dot(q, x_ref[...], preferred_element_type=jnp.float32)

    @pl.when(j == 0)
    def _init():
        acc_ref[...] = partial

    @pl.when(j > 0)
    def _accum():
        acc_ref[...] += partial

    @pl.when(j == nj - 1)
    def _epilogue():
        h = acc_ref[...]
        h = jnp.where(h >= 0, h, alpha_ref[...] * h)
        o_ref[...] = jax.lax.dot_general(
            h, W_ref[...], (((1,), (1,)), ((), ())),
            preferred_element_type=jnp.float32) + b_ref[...]


def kernel(theta, T_slices, x, a, prelu_alpha, W, b):
    grid = (_N // _BI, _N // _BJ)
    out = pl.pallas_call(
        _fused_kernel,
        grid=grid,
        in_specs=[
            pl.BlockSpec(memory_space=pltpu.SMEM),
            pl.BlockSpec((_K, _BI, _BJ), lambda i, j: (0, i, j)),
            pl.BlockSpec((_BI, _BJ), lambda i, j: (i, j)),
            pl.BlockSpec((_BJ, _D), lambda i, j: (j, 0)),
            pl.BlockSpec((1, _D), lambda i, j: (0, 0)),
            pl.BlockSpec((_D, _D), lambda i, j: (0, 0)),
            pl.BlockSpec((1, _D), lambda i, j: (0, 0)),
        ],
        out_specs=pl.BlockSpec((_BI, _D), lambda i, j: (i, 0)),
        out_shape=jax.ShapeDtypeStruct((_N, _D), jnp.float32),
        scratch_shapes=[pltpu.VMEM((_BI, _D), jnp.float32)],
        compiler_params=pltpu.CompilerParams(
            dimension_semantics=("parallel", "arbitrary"),
        ),
    )(theta, T_slices, a, x, prelu_alpha.reshape(1, _D), W,
      b.reshape(1, _D))
    return out


# speedup vs baseline: 1.1537x; 1.1537x over previous
import jax
import jax.numpy as jnp
from jax.experimental import pallas as pl
from jax.experimental.pallas import tpu as pltpu

_K, _N, _D = 8, 2048, 128
_BI = 128
_BJ = 2048


def _fused_kernel(theta_ref, T_ref, a_ref, x_ref, alpha_ref, W_ref, b_ref,
                  o_ref, acc_ref):
    j = pl.program_id(1)
    nj = pl.num_programs(1)

    parts = [theta_ref[k] * T_ref[k] for k in range(_K)]
    while len(parts) > 1:
        parts = [parts[i] + parts[i + 1] for i in range(0, len(parts), 2)]
    q = parts[0] * a_ref[...]

    partial = jnp.dot(q, x_ref[...], preferred_element_type=jnp.float32)

    @pl.when(j == 0)
    def _init():
        acc_ref[...] = partial

    @pl.when(j > 0)
    def _accum():
        acc_ref[...] += partial

    @pl.when(j == nj - 1)
    def _epilogue():
        h = acc_ref[...]
        h = jnp.where(h >= 0, h, alpha_ref[...] * h)
        o_ref[...] = jax.lax.dot_general(
            h, W_ref[...], (((1,), (1,)), ((), ())),
            preferred_element_type=jnp.float32) + b_ref[...]


def kernel(theta, T_slices, x, a, prelu_alpha, W, b):
    grid = (_N // _BI, _N // _BJ)
    out = pl.pallas_call(
        _fused_kernel,
        grid=grid,
        in_specs=[
            pl.BlockSpec(memory_space=pltpu.SMEM),
            pl.BlockSpec((_K, _BI, _BJ), lambda i, j: (0, i, j)),
            pl.BlockSpec((_BI, _BJ), lambda i, j: (i, j)),
            pl.BlockSpec((_BJ, _D), lambda i, j: (j, 0)),
            pl.BlockSpec((1, _D), lambda i, j: (0, 0)),
            pl.BlockSpec((_D, _D), lambda i, j: (0, 0)),
            pl.BlockSpec((1, _D), lambda i, j: (0, 0)),
        ],
        out_specs=pl.BlockSpec((_BI, _D), lambda i, j: (i, 0)),
        out_shape=jax.ShapeDtypeStruct((_N, _D), jnp.float32),
        scratch_shapes=[pltpu.VMEM((_BI, _D), jnp.float32)],
        compiler_params=pltpu.CompilerParams(
            dimension_semantics=("parallel", "arbitrary"),
        ),
    )(theta, T_slices, a, x, prelu_alpha.reshape(1, _D), W,
      b.reshape(1, _D))
    return out
